# Initial kernel scaffold; baseline (speedup 1.0000x reference)
import jax
import jax.numpy as jnp
from jax.experimental import pallas as pl


def kernel(z_e, embed_weight):
    raise NotImplementedError("write your pallas kernel here")


# speedup vs baseline: 1.0768x; 1.0768x over previous
import functools

import jax
import jax.numpy as jnp
from jax import lax
from jax.experimental import pallas as pl
from jax.experimental.pallas import tpu as pltpu
from jax.experimental.pallas import tpu_sc as plsc

_TOKENS = 16 * 1024
_DIM = 64
_CODES = 1024

_ROWS_PER_BLOCK = 1024
_NUM_BLOCKS = _TOKENS // _ROWS_PER_BLOCK

_NUM_WORKERS = 32
_ROWS_PER_WORKER = _TOKENS // _NUM_WORKERS


def _argmin_body(z_ref, et_ref, ind_ref, acc_ref):
    z = z_ref[...]
    et = et_ref[...]
    zg = jnp.dot(z, et, preferred_element_type=jnp.float32)
    rn = jnp.sum(z * z, axis=1, keepdims=True)
    en = jnp.sum(et * et, axis=0, keepdims=True)
    dist = rn - 2.0 * zg + en
    neg = -dist
    m = jnp.max(neg, axis=1, keepdims=True)
    ids = lax.broadcasted_iota(jnp.int32, neg.shape, 1)
    ind = jnp.min(jnp.where(neg == m, ids, jnp.int32(_CODES)), axis=1)
    ind_ref[0, 0, :] = ind

    @pl.when(pl.program_id(0) == 0)
    def _init():
        acc_ref[0, 0] = 0.0

    acc_ref[0, 0] += jnp.sum(-m)


def _tc_argmin(z_flat, et):
    return pl.pallas_call(
        _argmin_body,
        grid=(_NUM_BLOCKS,),
        in_specs=[
            pl.BlockSpec((_ROWS_PER_BLOCK, _DIM), lambda i: (i, 0)),
            pl.BlockSpec((_DIM, _CODES), lambda i: (0, 0)),
        ],
        out_specs=[
            pl.BlockSpec((1, 1, _ROWS_PER_BLOCK), lambda i: (i, 0, 0)),
            pl.BlockSpec((1, 1), lambda i: (0, 0), memory_space=pltpu.SMEM),
        ],
        out_shape=[
            jax.ShapeDtypeStruct((_NUM_BLOCKS, 1, _ROWS_PER_BLOCK), jnp.int32),
            jax.ShapeDtypeStruct((1, 1), jnp.float32),
        ],
    )(z_flat, et)


@functools.cache
def _build_sc_gather():
    @functools.partial(
        pl.kernel,
        out_type=jax.ShapeDtypeStruct((_TOKENS, _DIM), jnp.float32),
        mesh=plsc.VectorSubcoreMesh(core_axis_name="c", subcore_axis_name="s"),
        scratch_types=[
            pltpu.VMEM((_ROWS_PER_WORKER,), jnp.int32),
            pltpu.VMEM((_ROWS_PER_WORKER, _DIM), jnp.float32),
            pltpu.SemaphoreType.DMA,
        ],
        compiler_params=pltpu.CompilerParams(use_tc_tiling_on_sc=False),
    )
    def _sc_gather(table_hbm, idx_hbm, out_hbm, idx_v, rows_v, sem):
        wid = lax.axis_index("s") * 2 + lax.axis_index("c")
        base = wid * _ROWS_PER_WORKER
        pltpu.sync_copy(idx_hbm.at[pl.ds(base, _ROWS_PER_WORKER)], idx_v)
        pltpu.async_copy(table_hbm.at[idx_v], rows_v, sem).wait()
        pltpu.sync_copy(rows_v, out_hbm.at[pl.ds(base, _ROWS_PER_WORKER)])

    return _sc_gather


def kernel(z_e, embed_weight):
    z_flat = z_e.reshape(-1, _DIM)
    et = embed_weight.T
    ind_blocks, dist_sum = _tc_argmin(z_flat, et)
    ind_flat = ind_blocks.reshape(-1)
    zq_flat = _build_sc_gather()(embed_weight, ind_flat)
    z_q = zq_flat.reshape(z_e.shape)
    diff = (1.25 / z_e.size) * dist_sum[0, 0]
    embed_ind = ind_flat.reshape(z_e.shape[:-1])
    return (z_q, diff, embed_ind)


# speedup vs baseline: 1.1838x; 1.0994x over previous
import functools

import jax
import jax.numpy as jnp
from jax import lax
from jax.experimental import pallas as pl
from jax.experimental.pallas import tpu as pltpu
from jax.experimental.pallas import tpu_sc as plsc

_TOKENS = 16 * 1024
_DIM = 64
_CODES = 1024

_ROWS_PER_BLOCK = 1024
_NUM_BLOCKS = _TOKENS // _ROWS_PER_BLOCK

_NUM_WORKERS = 32
_ROWS_PER_WORKER = _TOKENS // _NUM_WORKERS


def _argmin_body(z_ref, et_ref, rev_ref, ind_ref, acc_ref):
    z = z_ref[...]
    et = et_ref[...]
    zg2 = jnp.dot(z + z, et, preferred_element_type=jnp.float32)
    rn = jnp.sum(z * z, axis=1, keepdims=True)
    en = jnp.sum(et * et, axis=0, keepdims=True)
    dist = (rn - zg2) + en
    m = jnp.min(dist, axis=1, keepdims=True)
    rev = rev_ref[...]
    picked = jnp.max(jnp.where(dist == m, rev, 0.0), axis=1)
    ind = _CODES - picked.astype(jnp.int32)
    ind_ref[0, 0, :] = ind

    @pl.when(pl.program_id(0) == 0)
    def _init():
        acc_ref[0, 0] = 0.0

    acc_ref[0, 0] += jnp.sum(m)


def _tc_argmin(z_flat, et, rev):
    return pl.pallas_call(
        _argmin_body,
        grid=(_NUM_BLOCKS,),
        in_specs=[
            pl.BlockSpec((_ROWS_PER_BLOCK, _DIM), lambda i: (i, 0)),
            pl.BlockSpec((_DIM, _CODES), lambda i: (0, 0)),
            pl.BlockSpec((1, _CODES), lambda i: (0, 0)),
        ],
        out_specs=[
            pl.BlockSpec((1, 1, _ROWS_PER_BLOCK), lambda i: (i, 0, 0)),
            pl.BlockSpec((1, 1), lambda i: (0, 0), memory_space=pltpu.SMEM),
        ],
        out_shape=[
            jax.ShapeDtypeStruct((_NUM_BLOCKS, 1, _ROWS_PER_BLOCK), jnp.int32),
            jax.ShapeDtypeStruct((1, 1), jnp.float32),
        ],
    )(z_flat, et, rev)


@functools.cache
def _build_sc_gather():
    @functools.partial(
        pl.kernel,
        out_type=jax.ShapeDtypeStruct((_TOKENS, _DIM), jnp.float32),
        mesh=plsc.VectorSubcoreMesh(core_axis_name="c", subcore_axis_name="s"),
        scratch_types=[
            pltpu.VMEM((_ROWS_PER_WORKER,), jnp.int32),
            pltpu.VMEM((_ROWS_PER_WORKER, _DIM), jnp.float32),
            pltpu.SemaphoreType.DMA,
        ],
        compiler_params=pltpu.CompilerParams(use_tc_tiling_on_sc=False),
    )
    def _sc_gather(table_hbm, idx_hbm, out_hbm, idx_v, rows_v, sem):
        wid = lax.axis_index("s") * 2 + lax.axis_index("c")
        base = wid * _ROWS_PER_WORKER
        pltpu.sync_copy(idx_hbm.at[pl.ds(base, _ROWS_PER_WORKER)], idx_v)
        pltpu.async_copy(table_hbm.at[idx_v], rows_v, sem).wait()
        pltpu.sync_copy(rows_v, out_hbm.at[pl.ds(base, _ROWS_PER_WORKER)])

    return _sc_gather


def kernel(z_e, embed_weight):
    z_flat = z_e.reshape(-1, _DIM)
    et = embed_weight.T
    rev = (_CODES - jnp.arange(_CODES, dtype=jnp.float32)).reshape(1, _CODES)
    ind_blocks, dist_sum = _tc_argmin(z_flat, et, rev)
    ind_flat = ind_blocks.reshape(-1)
    zq_flat = _build_sc_gather()(embed_weight, ind_flat)
    z_q = zq_flat.reshape(z_e.shape)
    diff = (1.25 / z_e.size) * dist_sum[0, 0]
    embed_ind = ind_flat.reshape(z_e.shape[:-1])
    return (z_q, diff, embed_ind)


# speedup vs baseline: 1.2064x; 1.0191x over previous
import functools

import jax
import jax.numpy as jnp
from jax import lax
from jax.experimental import pallas as pl
from jax.experimental.pallas import tpu as pltpu
from jax.experimental.pallas import tpu_sc as plsc

_TOKENS = 16 * 1024
_DIM = 64
_CODES = 1024

_ROWS_PER_BLOCK = 2048
_NUM_BLOCKS = _TOKENS // _ROWS_PER_BLOCK

_NUM_WORKERS = 32
_ROWS_PER_WORKER = _TOKENS // _NUM_WORKERS


def _argmin_body(z_ref, et_ref, rev_ref, ind_ref, acc_ref):
    z = z_ref[...]
    et = et_ref[...]
    zg2 = jnp.dot(z + z, et, preferred_element_type=jnp.float32)
    rn = jnp.sum(z * z, axis=1, keepdims=True)
    en = jnp.sum(et * et, axis=0, keepdims=True)
    dist = (rn - zg2) + en
    m = jnp.min(dist, axis=1, keepdims=True)
    rev = rev_ref[...]
    picked = jnp.max(jnp.where(dist == m, rev, 0.0), axis=1)
    ind = _CODES - picked.astype(jnp.int32)
    ind_ref[0, 0, :] = ind

    @pl.when(pl.program_id(0) == 0)
    def _init():
        acc_ref[0, 0] = 0.0

    acc_ref[0, 0] += jnp.sum(m)


def _tc_argmin(z_flat, et, rev):
    return pl.pallas_call(
        _argmin_body,
        grid=(_NUM_BLOCKS,),
        in_specs=[
            pl.BlockSpec((_ROWS_PER_BLOCK, _DIM), lambda i: (i, 0)),
            pl.BlockSpec((_DIM, _CODES), lambda i: (0, 0)),
            pl.BlockSpec((1, _CODES), lambda i: (0, 0)),
        ],
        out_specs=[
            pl.BlockSpec((1, 1, _ROWS_PER_BLOCK), lambda i: (i, 0, 0)),
            pl.BlockSpec((1, 1), lambda i: (0, 0), memory_space=pltpu.SMEM),
        ],
        out_shape=[
            jax.ShapeDtypeStruct((_NUM_BLOCKS, 1, _ROWS_PER_BLOCK), jnp.int32),
            jax.ShapeDtypeStruct((1, 1), jnp.float32),
        ],
    )(z_flat, et, rev)


@functools.cache
def _build_sc_gather():
    @functools.partial(
        pl.kernel,
        out_type=jax.ShapeDtypeStruct((_TOKENS, _DIM), jnp.float32),
        mesh=plsc.VectorSubcoreMesh(core_axis_name="c", subcore_axis_name="s"),
        scratch_types=[
            pltpu.VMEM((_ROWS_PER_WORKER,), jnp.int32),
            pltpu.VMEM((_ROWS_PER_WORKER, _DIM), jnp.float32),
            pltpu.SemaphoreType.DMA,
        ],
        compiler_params=pltpu.CompilerParams(use_tc_tiling_on_sc=False),
    )
    def _sc_gather(table_hbm, idx_hbm, out_hbm, idx_v, rows_v, sem):
        wid = lax.axis_index("s") * 2 + lax.axis_index("c")
        base = wid * _ROWS_PER_WORKER
        pltpu.sync_copy(idx_hbm.at[pl.ds(base, _ROWS_PER_WORKER)], idx_v)
        pltpu.async_copy(table_hbm.at[idx_v], rows_v, sem).wait()
        pltpu.sync_copy(rows_v, out_hbm.at[pl.ds(base, _ROWS_PER_WORKER)])

    return _sc_gather


def kernel(z_e, embed_weight):
    z_flat = z_e.reshape(-1, _DIM)
    et = embed_weight.T
    rev = (_CODES - jnp.arange(_CODES, dtype=jnp.float32)).reshape(1, _CODES)
    ind_blocks, dist_sum = _tc_argmin(z_flat, et, rev)
    ind_flat = ind_blocks.reshape(-1)
    zq_flat = _build_sc_gather()(embed_weight, ind_flat)
    z_q = zq_flat.reshape(z_e.shape)
    diff = (1.25 / z_e.size) * dist_sum[0, 0]
    embed_ind = ind_flat.reshape(z_e.shape[:-1])
    return (z_q, diff, embed_ind)


# speedup vs baseline: 1.2230x; 1.0138x over previous
import functools

import jax
import jax.numpy as jnp
from jax import lax
from jax.experimental import pallas as pl
from jax.experimental.pallas import tpu as pltpu
from jax.experimental.pallas import tpu_sc as plsc

_B = 16
_T = 1024
_TOKENS = _B * _T
_DIM = 64
_CODES = 1024

_ROWS_PER_BLOCK = 2048
_NUM_BLOCKS = _TOKENS // _ROWS_PER_BLOCK
_BATCH_PER_BLOCK = _ROWS_PER_BLOCK // _T

_NUM_WORKERS = 32
_ROWS_PER_WORKER = _TOKENS // _NUM_WORKERS
_SPLIT = _T // _ROWS_PER_WORKER


def _argmin_body(z_ref, e_ref, rev_ref, ind_ref, acc_ref):
    z = z_ref[...].reshape(_ROWS_PER_BLOCK, _DIM)
    e = e_ref[...]
    zg2 = lax.dot_general(
        z + z, e, (((1,), (1,)), ((), ())),
        preferred_element_type=jnp.float32,
    )
    rn = jnp.sum(z * z, axis=1, keepdims=True)
    en = jnp.sum(e * e, axis=1, keepdims=True).T
    dist = (rn - zg2) + en
    m = jnp.min(dist, axis=1, keepdims=True)
    rev = rev_ref[...]
    picked = jnp.max(jnp.where(dist == m, rev, 0.0), axis=1)
    ind = _CODES - picked.astype(jnp.int32)
    ind_ref[...] = ind.reshape(_BATCH_PER_BLOCK, 1, _T)

    @pl.when(pl.program_id(0) == 0)
    def _init():
        acc_ref[0, 0] = 0.0

    acc_ref[0, 0] += jnp.sum(m)


def _tc_argmin(z_e, embed_weight, rev):
    return pl.pallas_call(
        _argmin_body,
        grid=(_NUM_BLOCKS,),
        in_specs=[
            pl.BlockSpec((_BATCH_PER_BLOCK, _T, _DIM), lambda i: (i, 0, 0)),
            pl.BlockSpec((_CODES, _DIM), lambda i: (0, 0)),
            pl.BlockSpec((1, _CODES), lambda i: (0, 0)),
        ],
        out_specs=[
            pl.BlockSpec((_BATCH_PER_BLOCK, 1, _T), lambda i: (i, 0, 0)),
            pl.BlockSpec((1, 1), lambda i: (0, 0), memory_space=pltpu.SMEM),
        ],
        out_shape=[
            jax.ShapeDtypeStruct((_B, 1, _T), jnp.int32),
            jax.ShapeDtypeStruct((1, 1), jnp.float32),
        ],
    )(z_e, embed_weight, rev)


@functools.cache
def _build_sc_gather():
    @functools.partial(
        pl.kernel,
        out_type=jax.ShapeDtypeStruct((_B, _T, _DIM), jnp.float32),
        mesh=plsc.VectorSubcoreMesh(core_axis_name="c", subcore_axis_name="s"),
        scratch_types=[
            pltpu.VMEM((_ROWS_PER_WORKER,), jnp.int32),
            pltpu.VMEM((_ROWS_PER_WORKER, _DIM), jnp.float32),
            pltpu.SemaphoreType.DMA,
        ],
        compiler_params=pltpu.CompilerParams(use_tc_tiling_on_sc=False),
    )
    def _sc_gather(table_hbm, idx_hbm, out_hbm, idx_v, rows_v, sem):
        wid = lax.axis_index("s") * 2 + lax.axis_index("c")
        base = wid * _ROWS_PER_WORKER
        pltpu.sync_copy(idx_hbm.at[pl.ds(base, _ROWS_PER_WORKER)], idx_v)
        pltpu.async_copy(table_hbm.at[idx_v], rows_v, sem).wait()
        b = wid // _SPLIT
        col = (wid % _SPLIT) * _ROWS_PER_WORKER
        pltpu.sync_copy(rows_v, out_hbm.at[b, pl.ds(col, _ROWS_PER_WORKER)])

    return _sc_gather


def kernel(z_e, embed_weight):
    rev = (_CODES - jnp.arange(_CODES, dtype=jnp.float32)).reshape(1, _CODES)
    ind3, dist_sum = _tc_argmin(z_e, embed_weight, rev)
    ind_flat = ind3.reshape(-1)
    z_q = _build_sc_gather()(embed_weight, ind_flat)
    diff = (1.25 / z_e.size) * dist_sum[0, 0]
    embed_ind = ind3.reshape(_B, _T)
    return (z_q, diff, embed_ind)


# speedup vs baseline: 1.5553x; 1.2717x over previous
import functools

import jax
import jax.numpy as jnp
from jax import lax
from jax.experimental import pallas as pl
from jax.experimental.pallas import tpu as pltpu
from jax.experimental.pallas import tpu_sc as plsc

_B = 16
_T = 1024
_TOKENS = _B * _T
_DIM = 64
_CODES = 1024

_NUM_WORKERS = 32
_ROWS_PER_WORKER = _TOKENS // _NUM_WORKERS
_SPLIT = _T // _ROWS_PER_WORKER


def _argmin_body(zt_ref, e_ref, rev_ref, ind_ref, acc_ref):
    zt = zt_ref[0]
    e = e_ref[...]
    zg2 = jnp.dot(e + e, zt, preferred_element_type=jnp.float32)
    rn = jnp.sum(zt * zt, axis=0, keepdims=True)
    en = jnp.sum(e * e, axis=1, keepdims=True)
    dist = (rn - zg2) + en
    m = jnp.min(dist, axis=0, keepdims=True)
    rev = rev_ref[...]
    picked = jnp.max(jnp.where(dist == m, rev, 0.0), axis=0)
    ind = _CODES - picked.astype(jnp.int32)
    ind_ref[...] = ind.reshape(1, 1, _T)

    @pl.when(pl.program_id(0) == 0)
    def _init():
        acc_ref[0, 0] = 0.0

    acc_ref[0, 0] += jnp.sum(m)


def _tc_argmin(zt, embed_weight, rev):
    return pl.pallas_call(
        _argmin_body,
        grid=(_B,),
        in_specs=[
            pl.BlockSpec((1, _DIM, _T), lambda i: (i, 0, 0)),
            pl.BlockSpec((_CODES, _DIM), lambda i: (0, 0)),
            pl.BlockSpec((_CODES, 1), lambda i: (0, 0)),
        ],
        out_specs=[
            pl.BlockSpec((1, 1, _T), lambda i: (i, 0, 0)),
            pl.BlockSpec((1, 1), lambda i: (0, 0), memory_space=pltpu.SMEM),
        ],
        out_shape=[
            jax.ShapeDtypeStruct((_B, 1, _T), jnp.int32),
            jax.ShapeDtypeStruct((1, 1), jnp.float32),
        ],
    )(zt, embed_weight, rev)


@functools.cache
def _build_sc_gather():
    @functools.partial(
        pl.kernel,
        out_type=jax.ShapeDtypeStruct((_B, _T, _DIM), jnp.float32),
        mesh=plsc.VectorSubcoreMesh(core_axis_name="c", subcore_axis_name="s"),
        scratch_types=[
            pltpu.VMEM((_ROWS_PER_WORKER,), jnp.int32),
            pltpu.VMEM((_ROWS_PER_WORKER, _DIM), jnp.float32),
            pltpu.SemaphoreType.DMA,
        ],
        compiler_params=pltpu.CompilerParams(use_tc_tiling_on_sc=False),
    )
    def _sc_gather(table_hbm, idx_hbm, out_hbm, idx_v, rows_v, sem):
        wid = lax.axis_index("s") * 2 + lax.axis_index("c")
        base = wid * _ROWS_PER_WORKER
        pltpu.sync_copy(idx_hbm.at[pl.ds(base, _ROWS_PER_WORKER)], idx_v)
        pltpu.async_copy(table_hbm.at[idx_v], rows_v, sem).wait()
        b = wid // _SPLIT
        col = (wid % _SPLIT) * _ROWS_PER_WORKER
        pltpu.sync_copy(rows_v, out_hbm.at[b, pl.ds(col, _ROWS_PER_WORKER)])

    return _sc_gather


def kernel(z_e, embed_weight):
    zt = lax.transpose(z_e, (0, 2, 1))
    rev = (_CODES - jnp.arange(_CODES, dtype=jnp.float32)).reshape(_CODES, 1)
    ind3, dist_sum = _tc_argmin(zt, embed_weight, rev)
    ind_flat = ind3.reshape(-1)
    z_q = _build_sc_gather()(embed_weight, ind_flat)
    diff = (1.25 / z_e.size) * dist_sum[0, 0]
    embed_ind = ind3.reshape(_B, _T)
    return (z_q, diff, embed_ind)
